# baseline (speedup 1.0000x reference)
import functools

import jax
import jax.numpy as jnp
from jax.experimental import pallas as pl
from jax.experimental.pallas import tpu as pltpu

KS = 15
XP = 16
S0 = 0.6
GN_EPS = 1e-5
LRELU = 0.1
F32 = jnp.float32


def _ceil_to(x, m):
    return (x + m - 1) // m * m


def _lrelu(x):
    return jnp.where(x >= 0, x, LRELU * x)


def _kpconv_body(pts_ref, q_ref, kx_ref, nf_ref, sc_ref, sh_ref, w_ref,
                 o_ref, acc_ref, *, K, C, B, G, P, inv_sigma):
    q = q_ref[...]
    acc_ref[...] = jnp.zeros(acc_ref.shape, F32)

    def kstep(k, _):
        nbp = pts_ref[k]
        acc = None
        for i in range(3):
            pd = nbp[:, i:i + 1] - q[:, i:i + 1]
            t = pd * pd - 2.0 * (pd * kx_ref[i:i + 1, :])
            acc = t if acc is None else acc + t
        sq = acc + kx_ref[3:4, :]
        dist = jnp.sqrt(jnp.maximum(sq, 1e-12))
        infl = jnp.maximum(0.0, 1.0 - dist * inv_sigma)
        nfa = _lrelu(nf_ref[k] * sc_ref[...] + sh_ref[...])
        if P > 1:
            nfa = jnp.concatenate([nfa] * P, axis=1)
        for g in range(G):
            e = jnp.repeat(infl[:, g * P:(g + 1) * P], C, axis=1)
            acc_ref[:, g * 128:(g + 1) * 128] += e * nfa
        return 0

    jax.lax.fori_loop(0, K, kstep, 0)
    o_ref[...] = jax.lax.dot_general(
        acc_ref[...], w_ref[...], dimension_numbers=(((1,), (0,)), ((), ())),
        preferred_element_type=F32)


def _pack_w(kp_W, P, G):
    return jnp.pad(kp_W, ((0, G * P - KS), (0, 0), (0, 0))).reshape(
        G * P * kp_W.shape[1], kp_W.shape[2])


def _kpconv_pallas(pts_g, q_pts, kx, nf_g, scale, shift, kp_W, sigma,
                   B=512):
    K, Np, _ = pts_g.shape
    C = nf_g.shape[2]
    D = kp_W.shape[2]
    P = 128 // C
    G = (KS + P - 1) // P
    w_packed = _pack_w(kp_W, P, G)
    grid = Np // B
    return pl.pallas_call(
        functools.partial(_kpconv_body, K=K, C=C, B=B, G=G, P=P,
                          inv_sigma=1.0 / sigma),
        grid=(grid,),
        in_specs=[
            pl.BlockSpec((K, B, XP), lambda i: (0, i, 0)),
            pl.BlockSpec((B, XP), lambda i: (i, 0)),
            pl.BlockSpec((8, XP), lambda i: (0, 0)),
            pl.BlockSpec((K, B, C), lambda i: (0, i, 0)),
            pl.BlockSpec((1, C), lambda i: (0, 0)),
            pl.BlockSpec((1, C), lambda i: (0, 0)),
            pl.BlockSpec((G * 128, D), lambda i: (0, 0)),
        ],
        out_specs=pl.BlockSpec((B, D), lambda i: (i, 0)),
        out_shape=jax.ShapeDtypeStruct((Np, D), F32),
        scratch_shapes=[
            pltpu.VMEM((B, G * 128), F32),
        ],
    )(pts_g, q_pts, kx, nf_g, scale, shift, w_packed)


def _e11_body(pts_ref, q_ref, kx_ref, w_ref, o_ref, *, K, B, inv_sigma):
    q = q_ref[...]

    def kstep(k, s):
        nbp = pts_ref[k]
        acc = None
        for i in range(3):
            pd = nbp[:, i:i + 1] - q[:, i:i + 1]
            t = pd * pd - 2.0 * (pd * kx_ref[i:i + 1, :])
            acc = t if acc is None else acc + t
        sq = acc + kx_ref[3:4, :]
        dist = jnp.sqrt(jnp.maximum(sq, 1e-12))
        return s + jnp.maximum(0.0, 1.0 - dist * inv_sigma)

    s = jax.lax.fori_loop(0, K, kstep, jnp.zeros((B, XP), F32))
    o_ref[...] = jax.lax.dot_general(
        s, w_ref[...], dimension_numbers=(((1,), (0,)), ((), ())),
        preferred_element_type=F32)


def _e11_pallas(pts_g, q_pts, kx, w0_pad, sigma, B=512):
    K, Np, _ = pts_g.shape
    D = w0_pad.shape[1]
    return pl.pallas_call(
        functools.partial(_e11_body, K=K, B=B, inv_sigma=1.0 / sigma),
        grid=(Np // B,),
        in_specs=[
            pl.BlockSpec((K, B, XP), lambda i: (0, i, 0)),
            pl.BlockSpec((B, XP), lambda i: (i, 0)),
            pl.BlockSpec((8, XP), lambda i: (0, 0)),
            pl.BlockSpec((XP, D), lambda i: (0, 0)),
        ],
        out_specs=pl.BlockSpec((B, D), lambda i: (i, 0)),
        out_shape=jax.ShapeDtypeStruct((Np, D), F32),
    )(pts_g, q_pts, kx, w0_pad)


def _scmax_body(scf_ref, o_ref, *, K):
    def kstep(k, acc):
        return jnp.maximum(acc, scf_ref[k])
    o_ref[...] = jax.lax.fori_loop(
        0, K, kstep, jnp.full(o_ref.shape, -jnp.inf, F32))


def _scmax_pallas(scf_g, B=512):
    K, Np, C = scf_g.shape
    return pl.pallas_call(
        functools.partial(_scmax_body, K=K),
        grid=(Np // B,),
        in_specs=[pl.BlockSpec((K, B, C), lambda i: (0, i, 0))],
        out_specs=pl.BlockSpec((B, C), lambda i: (i, 0)),
        out_shape=jax.ShapeDtypeStruct((Np, C), F32),
    )(scf_g)


def _mm_body(x_ref, w_ref, sc_ref, sh_ref, o_ref, *, fuse_act):
    x = x_ref[...]
    if fuse_act:
        x = _lrelu(x * sc_ref[...] + sh_ref[...])
    o_ref[...] = jax.lax.dot_general(
        x, w_ref[...], dimension_numbers=(((1,), (0,)), ((), ())),
        preferred_element_type=F32)


def _mm_pallas(x, w, scale=None, shift=None, B=1024):
    n_in = x.shape[0]
    Np = _ceil_to(n_in, B)
    if Np != n_in:
        x = jnp.pad(x, ((0, Np - n_in), (0, 0)))
    Cin = x.shape[1]
    D = w.shape[1]
    fuse = scale is not None
    if not fuse:
        scale = jnp.ones((1, Cin), F32)
        shift = jnp.zeros((1, Cin), F32)
    else:
        scale = scale.reshape(1, Cin)
        shift = shift.reshape(1, Cin)
    return pl.pallas_call(
        functools.partial(_mm_body, fuse_act=fuse),
        grid=(Np // B,),
        in_specs=[
            pl.BlockSpec((B, Cin), lambda i: (i, 0)),
            pl.BlockSpec((Cin, D), lambda i: (0, 0)),
            pl.BlockSpec((1, Cin), lambda i: (0, 0)),
            pl.BlockSpec((1, Cin), lambda i: (0, 0)),
        ],
        out_specs=pl.BlockSpec((B, D), lambda i: (i, 0)),
        out_shape=jax.ShapeDtypeStruct((Np, D), F32),
    )(x, w, scale, shift)


def _gn_affine(x_valid, gamma, beta, groups=8):
    n, c = x_valid.shape
    gs = c // groups
    xg = x_valid.reshape(n, groups, gs)
    mean = xg.mean(axis=(0, 2))
    var = xg.var(axis=(0, 2))
    rs = jax.lax.rsqrt(var + GN_EPS)
    scale = jnp.repeat(rs, gs) * gamma
    shift = beta - jnp.repeat(mean * rs, gs) * gamma
    return scale, shift


def _gather_kmajor(table, neighbors_t, Np):
    K, N = neighbors_t.shape
    g = table[neighbors_t]
    if Np != N:
        g = jnp.pad(g, ((0, 0), (0, Np - N), (0, 0)))
    return g


def _pts_pad(points, Np):
    N = points.shape[0]
    return jnp.pad(points, ((0, Np - N), (0, XP - 3)))


def _kx_const(kpts):
    kx = jnp.zeros((8, XP), F32)
    kx = kx.at[0:3, :KS].set(kpts.T)
    kx = kx.at[3, :KS].set((kpts * kpts).sum(-1))
    return kx


def _res_block(p, s_feats, pts_g, q_pts, nb_t, sigma, Np_q, Np_s, strided):
    N_q = nb_t.shape[1]
    N_s = s_feats.shape[0]
    x = _mm_pallas(s_feats, p['u1_W'])[:N_s]
    sc1, sh1 = _gn_affine(x, p['u1_g'], p['u1_b'])
    nf_g = _gather_kmajor(x, nb_t, Np_q)
    kx = _kx_const(p['kpts'])
    kp = _kpconv_pallas(pts_g, q_pts, kx, nf_g,
                        sc1.reshape(1, -1), sh1.reshape(1, -1),
                        p['kp_W'], sigma)[:N_q]
    sc2, sh2 = _gn_affine(kp, p['kn_g'], p['kn_b'])
    y = _mm_pallas(kp, p['u2_W'], scale=sc2, shift=sh2)[:N_q]
    sc3, sh3 = _gn_affine(y, p['u2_g'], p['u2_b'])
    y = y * sc3 + sh3
    if strided:
        scf_g = _gather_kmajor(s_feats, nb_t, Np_q)
        sc = _scmax_pallas(scf_g)[:N_q]
    else:
        sc = s_feats
    if 'sc_W' in p:
        sc = _mm_pallas(sc, p['sc_W'])[:N_q]
        sc4, sh4 = _gn_affine(sc, p['sc_g'], p['sc_b'])
        sc = sc * sc4 + sh4
    return _lrelu(y + sc)


def kernel(points_0, points_1, points_2, neighbors_0, neighbors_1,
           neighbors_2, subsampling_0, subsampling_1, upsampling_0, params):
    N0 = points_0.shape[0]
    N1 = points_1.shape[0]
    N2 = points_2.shape[0]
    B = 512
    Np0, Np1, Np2 = _ceil_to(N0, B), _ceil_to(N1, B), _ceil_to(N2, B)
    p = params

    pp0 = _pts_pad(points_0, Np0)
    pp1 = _pts_pad(points_1, Np1)
    pp2 = _pts_pad(points_2, Np2)

    nb0_t = neighbors_0.T
    nb1_t = neighbors_1.T
    nb2_t = neighbors_2.T
    ss0_t = subsampling_0.T
    ss1_t = subsampling_1.T

    g_nb0 = _gather_kmajor(pp0[:N0, :], nb0_t, Np0)
    g_ss0 = _gather_kmajor(pp0[:N0, :], ss0_t, Np1)
    g_nb1 = _gather_kmajor(pp1[:N1, :], nb1_t, Np1)
    g_ss1 = _gather_kmajor(pp1[:N1, :], ss1_t, Np2)
    g_nb2 = _gather_kmajor(pp2[:N2, :], nb2_t, Np2)

    w0 = jnp.pad(p['e11']['kp_W'][:, 0, :], ((0, XP - KS), (0, 0)))
    feats = _e11_pallas(g_nb0, pp0, _kx_const(p['e11']['kpts']), w0, S0)[:N0]
    sc, sh = _gn_affine(feats, p['e11']['g'], p['e11']['b'])
    feats = _lrelu(feats * sc + sh)

    feats = _res_block(p['e12'], feats, g_nb0, pp0, nb0_t, S0, Np0, Np0,
                       strided=False)
    feats = _res_block(p['l1_0'], feats, g_ss0, pp1, ss0_t, S0, Np1, Np0,
                       strided=True)
    feats = _res_block(p['l1_1'], feats, g_nb1, pp1, nb1_t, 2 * S0, Np1, Np1,
                       strided=False)
    f1 = _res_block(p['l1_2'], feats, g_nb1, pp1, nb1_t, 2 * S0, Np1, Np1,
                    strided=False)
    feats = _res_block(p['l2_0'], f1, g_ss1, pp2, ss1_t, 2 * S0, Np2, Np1,
                       strided=True)
    feats = _res_block(p['l2_1'], feats, g_nb2, pp2, nb2_t, 4 * S0, Np2, Np2,
                       strided=False)
    f2 = _res_block(p['l2_2'], feats, g_nb2, pp2, nb2_t, 4 * S0, Np2, Np2,
                    strided=False)

    up = jnp.concatenate([f1, f2[upsampling_0[:, 0]]], axis=1)
    f1d = _mm_pallas(up, p['dec0_W'])[:N1]
    scd, shd = _gn_affine(f1d, p['dec0_g'], p['dec0_b'])
    f1d = _lrelu(f1d * scd + shd)

    d2 = ((points_2[:, None, :] - points_1[None, :, :]) ** 2).sum(-1)
    _unused, idx = jax.lax.top_k(-d2, 32)
    gx = points_1[idx]
    gf = f1d[idx]
    rel = gx - points_2[:, None, :]
    h = _lrelu(jnp.concatenate([rel, gf], axis=-1) @ p['det_W1']
               + p['det_b1'])
    scores = (h @ p['det_W2'] + p['det_b2'])[..., 0]
    attn = jax.nn.softmax(scores, axis=-1)
    xyz = jnp.einsum('mk,mki->mi', attn, gx)
    dist = jnp.sqrt(((gx - xyz[:, None, :]) ** 2).sum(-1) + 1e-12)
    sigma_out = jnp.einsum('mk,mk->m', attn, dist)[:, None]
    att_feat = jnp.einsum('mk,mkc->mc', attn, gf)
    g = jnp.max(_lrelu(gf @ p['desc_Wg']), axis=1)
    a = att_feat @ p['desc_Wa']
    desc = jnp.concatenate([g, a], axis=-1) @ p['desc_Wo']
    desc = desc / (jnp.linalg.norm(desc, axis=-1, keepdims=True) + 1e-8)
    return (f1d, f2, xyz, sigma_out, desc)


# speedup vs baseline: 2.9034x; 2.9034x over previous
import functools

import jax
import jax.numpy as jnp
from jax.experimental import pallas as pl
from jax.experimental.pallas import tpu as pltpu
from jax.experimental.pallas import tpu_sc as plsc

KS = 15
XP = 16
S0 = 0.6
GN_EPS = 1e-5
LRELU = 0.1
F32 = jnp.float32


def _ceil_to(x, m):
    return (x + m - 1) // m * m


def _lrelu(x):
    return jnp.where(x >= 0, x, LRELU * x)


def _dot(a, b):
    return jax.lax.dot_general(
        a, b, dimension_numbers=(((1,), (0,)), ((), ())),
        preferred_element_type=F32,
        precision=jax.lax.Precision.HIGHEST)


def _infl_mxu(nbp, q, m2_ref, kn, inv_sigma):
    dw = nbp - q
    u = jnp.concatenate([dw, dw * dw], axis=1)
    sq = _dot(u, m2_ref) + kn
    dist = jnp.sqrt(jnp.maximum(sq, 1e-12))
    return jnp.maximum(0.0, 1.0 - dist * inv_sigma)


def _kpconv_body(pts_ref, q_ref, m2_ref, kn_ref, nf_ref, mu_ref, sc_ref,
                 w_ref, o_ref, infl_s, nfa_s, acc_s, *, K, C, B, G, P,
                 inv_sigma):
    q = q_ref[...]
    kn = kn_ref[...]

    def phase1(k, _):
        infl_s[k] = _infl_mxu(pts_ref[k], q, m2_ref[...], kn, inv_sigma)
        nfa = _lrelu((nf_ref[k] - mu_ref[...]) * sc_ref[...])
        if P > 1:
            nfa = jnp.concatenate([nfa] * P, axis=1)
        nfa_s[k] = nfa
        return 0

    jax.lax.fori_loop(0, K, phase1, 0)

    for g in range(G):
        def gstep(k, acc, g=g):
            e = jnp.repeat(infl_s[k][:, g * P:(g + 1) * P], C, axis=1)
            return acc + e * nfa_s[k]
        acc_s[:, g * 128:(g + 1) * 128] = jax.lax.fori_loop(
            0, K, gstep, jnp.zeros((B, P * C), F32))
    o_ref[...] = _dot(acc_s[...], w_ref[...])


def _pack_w(kp_W, P, G):
    return jnp.pad(kp_W, ((0, G * P - KS), (0, 0), (0, 0))).reshape(
        G * P * kp_W.shape[1], kp_W.shape[2])


def _kpconv_pallas(pts_g, q_pts, m2, kn, nf_g, mu, scale, kp_W, sigma,
                   B=None):
    K, Np, _ = pts_g.shape
    C = nf_g.shape[2]
    if B is None:
        B = 512 if C <= 64 else 256
    D = kp_W.shape[2]
    P = 128 // C
    G = (KS + P - 1) // P
    w_packed = _pack_w(kp_W, P, G)
    grid = Np // B
    return pl.pallas_call(
        functools.partial(_kpconv_body, K=K, C=C, B=B, G=G, P=P,
                          inv_sigma=1.0 / sigma),
        grid=(grid,),
        in_specs=[
            pl.BlockSpec((K, B, XP), lambda i: (0, i, 0)),
            pl.BlockSpec((B, XP), lambda i: (i, 0)),
            pl.BlockSpec((2 * XP, XP), lambda i: (0, 0)),
            pl.BlockSpec((1, XP), lambda i: (0, 0)),
            pl.BlockSpec((K, B, C), lambda i: (0, i, 0)),
            pl.BlockSpec((1, C), lambda i: (0, 0)),
            pl.BlockSpec((1, C), lambda i: (0, 0)),
            pl.BlockSpec((G * 128, D), lambda i: (0, 0)),
        ],
        out_specs=pl.BlockSpec((B, D), lambda i: (i, 0)),
        out_shape=jax.ShapeDtypeStruct((Np, D), F32),
        scratch_shapes=[
            pltpu.VMEM((K, B, XP), F32),
            pltpu.VMEM((K, B, P * C), F32),
            pltpu.VMEM((B, G * 128), F32),
        ],
    )(pts_g, q_pts, m2, kn, nf_g, mu, scale, w_packed)


def _e11_body(pts_ref, q_ref, m2_ref, kn_ref, w_ref, o_ref, *, K, B,
              inv_sigma):
    q = q_ref[...]
    kn = kn_ref[...]

    def kstep(k, s):
        return s + _infl_mxu(pts_ref[k], q, m2_ref[...], kn, inv_sigma)

    s = jax.lax.fori_loop(0, K, kstep, jnp.zeros((B, XP), F32))
    o_ref[...] = _dot(s, w_ref[...])


def _e11_pallas(pts_g, q_pts, m2, kn, w0_pad, sigma, B=512):
    K, Np, _ = pts_g.shape
    D = w0_pad.shape[1]
    return pl.pallas_call(
        functools.partial(_e11_body, K=K, B=B, inv_sigma=1.0 / sigma),
        grid=(Np // B,),
        in_specs=[
            pl.BlockSpec((K, B, XP), lambda i: (0, i, 0)),
            pl.BlockSpec((B, XP), lambda i: (i, 0)),
            pl.BlockSpec((2 * XP, XP), lambda i: (0, 0)),
            pl.BlockSpec((1, XP), lambda i: (0, 0)),
            pl.BlockSpec((XP, D), lambda i: (0, 0)),
        ],
        out_specs=pl.BlockSpec((B, D), lambda i: (i, 0)),
        out_shape=jax.ShapeDtypeStruct((Np, D), F32),
    )(pts_g, q_pts, m2, kn, w0_pad)


def _scmax_body(scf_ref, o_ref, *, K):
    def kstep(k, acc):
        return jnp.maximum(acc, scf_ref[k])
    o_ref[...] = jax.lax.fori_loop(
        0, K, kstep, jnp.full(o_ref.shape, -jnp.inf, F32))


def _scmax_pallas(scf_g, B=512):
    K, Np, C = scf_g.shape
    return pl.pallas_call(
        functools.partial(_scmax_body, K=K),
        grid=(Np // B,),
        in_specs=[pl.BlockSpec((K, B, C), lambda i: (0, i, 0))],
        out_specs=pl.BlockSpec((B, C), lambda i: (i, 0)),
        out_shape=jax.ShapeDtypeStruct((Np, C), F32),
    )(scf_g)


def _mm_body(x_ref, w_ref, mu_ref, sc_ref, o_ref, *, fuse_act):
    x = x_ref[...]
    if fuse_act:
        x = _lrelu((x - mu_ref[...]) * sc_ref[...])
    o_ref[...] = _dot(x, w_ref[...])


def _mm_pallas(x, w, mu=None, scale=None, B=1024):
    n_in = x.shape[0]
    Np = _ceil_to(n_in, B)
    if Np != n_in:
        x = jnp.pad(x, ((0, Np - n_in), (0, 0)))
    Cin = x.shape[1]
    D = w.shape[1]
    fuse = scale is not None
    if not fuse:
        mu = jnp.zeros((1, Cin), F32)
        scale = jnp.ones((1, Cin), F32)
    else:
        mu = mu.reshape(1, Cin)
        scale = scale.reshape(1, Cin)
    return pl.pallas_call(
        functools.partial(_mm_body, fuse_act=fuse),
        grid=(Np // B,),
        in_specs=[
            pl.BlockSpec((B, Cin), lambda i: (i, 0)),
            pl.BlockSpec((Cin, D), lambda i: (0, 0)),
            pl.BlockSpec((1, Cin), lambda i: (0, 0)),
            pl.BlockSpec((1, Cin), lambda i: (0, 0)),
        ],
        out_specs=pl.BlockSpec((B, D), lambda i: (i, 0)),
        out_shape=jax.ShapeDtypeStruct((Np, D), F32),
    )(x, w, mu, scale)


def _gn_affine(x_valid, gamma, beta, groups=8):
    n, c = x_valid.shape
    gs = c // groups
    xg = x_valid.reshape(n, groups, gs)
    mean = jnp.repeat(xg.mean(axis=(0, 2)), gs)
    var = jnp.repeat(xg.var(axis=(0, 2)), gs)
    scale = jax.lax.rsqrt(var + GN_EPS) * gamma
    safe = jnp.where(scale == 0, 1.0, scale)
    mu_eff = mean - jnp.where(scale == 0, 0.0, beta / safe)
    return mu_eff, scale


def _gather_kmajor(table, neighbors_t, Np):
    K, N = neighbors_t.shape
    idx = neighbors_t if Np == N else jnp.pad(neighbors_t,
                                              ((0, 0), (0, Np - N)))
    Rn = K * Np
    idx2 = idx.reshape(1, Rn).astype(jnp.int32)
    Wd = table.shape[1]
    win = 128
    mesh = plsc.VectorSubcoreMesh(core_axis_name="c", subcore_axis_name="s")
    kw = {}
    if Wd % 128 != 0:
        kw['compiler_params'] = pltpu.CompilerParams(
            use_tc_tiling_on_sc=False)

    @functools.partial(pl.kernel,
                       out_type=jax.ShapeDtypeStruct((Rn, Wd), F32),
                       mesh=mesh, **kw)
    def gk(table_hbm, i_hbm, o_hbm):
        def body(i_vmem, o_vmem):
            pltpu.sync_copy(table_hbm.at[i_vmem.at[0]], o_vmem)

        pltpu.emit_pipeline(
            body,
            grid=(Rn // win,),
            in_specs=[pl.BlockSpec((1, win), index_map=lambda i: (0, i))],
            out_specs=[pl.BlockSpec((win, Wd), index_map=lambda i: (i, 0))],
            core_axis_name=("c", "s"),
            dimension_semantics=(pltpu.PARALLEL,),
        )(i_hbm, o_hbm)

    return gk(table, idx2).reshape(K, Np, Wd)


def _pts_pad(points, Np):
    N = points.shape[0]
    return jnp.pad(points, ((0, Np - N), (0, XP - 3)))


def _m2_const(kpts):
    m2 = jnp.zeros((2 * XP, XP), F32)
    m2 = m2.at[0:3, :KS].set(-2.0 * kpts.T)
    m2 = m2.at[XP:XP + 3, :].set(1.0)
    return m2


def _kn_const(kpts):
    kn = jnp.zeros((1, XP), F32)
    return kn.at[0, :KS].set((kpts * kpts).sum(-1))


def _res_block(p, s_feats, pts_g, q_pts, nb_t, sigma, Np_q, Np_s, strided):
    N_q = nb_t.shape[1]
    N_s = s_feats.shape[0]
    x = _mm_pallas(s_feats, p['u1_W'])[:N_s]
    mu1, sc1 = _gn_affine(x, p['u1_g'], p['u1_b'])
    nf_g = _gather_kmajor(x, nb_t, Np_q)
    kp = _kpconv_pallas(pts_g, q_pts, _m2_const(p['kpts']),
                        _kn_const(p['kpts']), nf_g,
                        mu1.reshape(1, -1), sc1.reshape(1, -1),
                        p['kp_W'], sigma)[:N_q]
    mu2, sc2 = _gn_affine(kp, p['kn_g'], p['kn_b'])
    y = _mm_pallas(kp, p['u2_W'], mu=mu2, scale=sc2)[:N_q]
    mu3, sc3 = _gn_affine(y, p['u2_g'], p['u2_b'])
    y = (y - mu3) * sc3
    if strided:
        scf_g = _gather_kmajor(s_feats, nb_t, Np_q)
        sc = _scmax_pallas(scf_g)[:N_q]
    else:
        sc = s_feats
    if 'sc_W' in p:
        sc = _mm_pallas(sc, p['sc_W'])[:N_q]
        mu4, sc4 = _gn_affine(sc, p['sc_g'], p['sc_b'])
        sc = (sc - mu4) * sc4
    return _lrelu(y + sc)


def kernel(points_0, points_1, points_2, neighbors_0, neighbors_1,
           neighbors_2, subsampling_0, subsampling_1, upsampling_0, params):
    N0 = points_0.shape[0]
    N1 = points_1.shape[0]
    N2 = points_2.shape[0]
    B = 512
    Np0, Np1, Np2 = _ceil_to(N0, B), _ceil_to(N1, B), _ceil_to(N2, B)
    p = params

    pp0 = _pts_pad(points_0, Np0)
    pp1 = _pts_pad(points_1, Np1)
    pp2 = _pts_pad(points_2, Np2)

    nb0_t = neighbors_0.T
    nb1_t = neighbors_1.T
    nb2_t = neighbors_2.T
    ss0_t = subsampling_0.T
    ss1_t = subsampling_1.T

    g_nb0 = _gather_kmajor(pp0[:N0, :], nb0_t, Np0)
    g_ss0 = _gather_kmajor(pp0[:N0, :], ss0_t, Np1)
    g_nb1 = _gather_kmajor(pp1[:N1, :], nb1_t, Np1)
    g_ss1 = _gather_kmajor(pp1[:N1, :], ss1_t, Np2)
    g_nb2 = _gather_kmajor(pp2[:N2, :], nb2_t, Np2)

    w0 = jnp.pad(p['e11']['kp_W'][:, 0, :], ((0, XP - KS), (0, 0)))
    feats = _e11_pallas(g_nb0, pp0, _m2_const(p['e11']['kpts']),
                        _kn_const(p['e11']['kpts']), w0, S0)[:N0]
    mu, sc = _gn_affine(feats, p['e11']['g'], p['e11']['b'])
    feats = _lrelu((feats - mu) * sc)

    feats = _res_block(p['e12'], feats, g_nb0, pp0, nb0_t, S0, Np0, Np0,
                       strided=False)
    feats = _res_block(p['l1_0'], feats, g_ss0, pp1, ss0_t, S0, Np1, Np0,
                       strided=True)
    feats = _res_block(p['l1_1'], feats, g_nb1, pp1, nb1_t, 2 * S0, Np1, Np1,
                       strided=False)
    f1 = _res_block(p['l1_2'], feats, g_nb1, pp1, nb1_t, 2 * S0, Np1, Np1,
                    strided=False)
    feats = _res_block(p['l2_0'], f1, g_ss1, pp2, ss1_t, 2 * S0, Np2, Np1,
                       strided=True)
    feats = _res_block(p['l2_1'], feats, g_nb2, pp2, nb2_t, 4 * S0, Np2, Np2,
                       strided=False)
    f2 = _res_block(p['l2_2'], feats, g_nb2, pp2, nb2_t, 4 * S0, Np2, Np2,
                    strided=False)

    up = jnp.concatenate([f1, f2[upsampling_0[:, 0]]], axis=1)
    f1d = _mm_pallas(up, p['dec0_W'])[:N1]
    mud, scd = _gn_affine(f1d, p['dec0_g'], p['dec0_b'])
    f1d = _lrelu((f1d - mud) * scd)

    d2 = ((points_2[:, None, :] - points_1[None, :, :]) ** 2).sum(-1)
    _unused, idx = jax.lax.top_k(-d2, 32)
    gx = points_1[idx]
    gf = f1d[idx]
    rel = gx - points_2[:, None, :]
    h = _lrelu(jnp.concatenate([rel, gf], axis=-1) @ p['det_W1']
               + p['det_b1'])
    scores = (h @ p['det_W2'] + p['det_b2'])[..., 0]
    attn = jax.nn.softmax(scores, axis=-1)
    xyz = jnp.einsum('mk,mki->mi', attn, gx)
    dist = jnp.sqrt(((gx - xyz[:, None, :]) ** 2).sum(-1) + 1e-12)
    sigma_out = jnp.einsum('mk,mk->m', attn, dist)[:, None]
    att_feat = jnp.einsum('mk,mkc->mc', attn, gf)
    g = jnp.max(_lrelu(gf @ p['desc_Wg']), axis=1)
    a = att_feat @ p['desc_Wa']
    desc = jnp.concatenate([g, a], axis=-1) @ p['desc_Wo']
    desc = desc / (jnp.linalg.norm(desc, axis=-1, keepdims=True) + 1e-8)
    return (f1d, f2, xyz, sigma_out, desc)


# speedup vs baseline: 3.0092x; 1.0364x over previous
import functools

import jax
import jax.numpy as jnp
from jax.experimental import pallas as pl
from jax.experimental.pallas import tpu as pltpu
from jax.experimental.pallas import tpu_sc as plsc

KS = 15
XP = 16
S0 = 0.6
GN_EPS = 1e-5
LRELU = 0.1
F32 = jnp.float32


def _ceil_to(x, m):
    return (x + m - 1) // m * m


def _lrelu(x):
    return jnp.where(x >= 0, x, LRELU * x)


def _dot(a, b):
    return jax.lax.dot_general(
        a, b, dimension_numbers=(((1,), (0,)), ((), ())),
        preferred_element_type=F32,
        precision=jax.lax.Precision.HIGHEST)


def _infl_mxu(nbp, q, m2_ref, kn, inv_sigma):
    dw = nbp - q
    u = jnp.concatenate([dw, dw * dw], axis=1)
    sq = _dot(u, m2_ref) + kn
    dist = jnp.sqrt(jnp.maximum(sq, 1e-12))
    return jnp.maximum(0.0, 1.0 - dist * inv_sigma)


def _kpconv_body(pts_ref, q_ref, m2_ref, kn_ref, nf_ref, mu_ref, sc_ref,
                 w_ref, o_ref, infl_s, nfa_s, acc_s, *, K, C, B, G, P,
                 inv_sigma):
    q = q_ref[...]
    kn = kn_ref[...]

    def phase1(k, _):
        infl_s[k] = _infl_mxu(pts_ref[k], q, m2_ref[...], kn, inv_sigma)
        nfa = _lrelu((nf_ref[k] - mu_ref[...]) * sc_ref[...])
        if P > 1:
            nfa = jnp.concatenate([nfa] * P, axis=1)
        nfa_s[k] = nfa
        return 0

    jax.lax.fori_loop(0, K, phase1, 0)

    for g in range(G):
        def gstep(k, acc, g=g):
            e = jnp.repeat(infl_s[k][:, g * P:(g + 1) * P], C, axis=1)
            return acc + e * nfa_s[k]
        acc_s[:, g * 128:(g + 1) * 128] = jax.lax.fori_loop(
            0, K, gstep, jnp.zeros((B, P * C), F32))
    o_ref[...] = _dot(acc_s[...], w_ref[...])


def _pack_w(kp_W, P, G):
    return jnp.pad(kp_W, ((0, G * P - KS), (0, 0), (0, 0))).reshape(
        G * P * kp_W.shape[1], kp_W.shape[2])


def _kpconv_pallas(pts_g, q_pts, m2, kn, nf_g, mu, scale, kp_W, sigma,
                   B=None):
    K, Np, _ = pts_g.shape
    C = nf_g.shape[2]
    if B is None:
        B = 512 if C <= 64 else 256
    D = kp_W.shape[2]
    P = 128 // C
    G = (KS + P - 1) // P
    w_packed = _pack_w(kp_W, P, G)
    grid = Np // B
    return pl.pallas_call(
        functools.partial(_kpconv_body, K=K, C=C, B=B, G=G, P=P,
                          inv_sigma=1.0 / sigma),
        grid=(grid,),
        in_specs=[
            pl.BlockSpec((K, B, XP), lambda i: (0, i, 0)),
            pl.BlockSpec((B, XP), lambda i: (i, 0)),
            pl.BlockSpec((2 * XP, XP), lambda i: (0, 0)),
            pl.BlockSpec((1, XP), lambda i: (0, 0)),
            pl.BlockSpec((K, B, C), lambda i: (0, i, 0)),
            pl.BlockSpec((1, C), lambda i: (0, 0)),
            pl.BlockSpec((1, C), lambda i: (0, 0)),
            pl.BlockSpec((G * 128, D), lambda i: (0, 0)),
        ],
        out_specs=pl.BlockSpec((B, D), lambda i: (i, 0)),
        out_shape=jax.ShapeDtypeStruct((Np, D), F32),
        scratch_shapes=[
            pltpu.VMEM((K, B, XP), F32),
            pltpu.VMEM((K, B, P * C), F32),
            pltpu.VMEM((B, G * 128), F32),
        ],
    )(pts_g, q_pts, m2, kn, nf_g, mu, scale, w_packed)


def _e11_body(pts_ref, q_ref, m2_ref, kn_ref, w_ref, o_ref, *, K, B,
              inv_sigma):
    q = q_ref[...]
    kn = kn_ref[...]

    def kstep(k, s):
        return s + _infl_mxu(pts_ref[k], q, m2_ref[...], kn, inv_sigma)

    s = jax.lax.fori_loop(0, K, kstep, jnp.zeros((B, XP), F32))
    o_ref[...] = _dot(s, w_ref[...])


def _e11_pallas(pts_g, q_pts, m2, kn, w0_pad, sigma, B=512):
    K, Np, _ = pts_g.shape
    D = w0_pad.shape[1]
    return pl.pallas_call(
        functools.partial(_e11_body, K=K, B=B, inv_sigma=1.0 / sigma),
        grid=(Np // B,),
        in_specs=[
            pl.BlockSpec((K, B, XP), lambda i: (0, i, 0)),
            pl.BlockSpec((B, XP), lambda i: (i, 0)),
            pl.BlockSpec((2 * XP, XP), lambda i: (0, 0)),
            pl.BlockSpec((1, XP), lambda i: (0, 0)),
            pl.BlockSpec((XP, D), lambda i: (0, 0)),
        ],
        out_specs=pl.BlockSpec((B, D), lambda i: (i, 0)),
        out_shape=jax.ShapeDtypeStruct((Np, D), F32),
    )(pts_g, q_pts, m2, kn, w0_pad)


def _scmax_body(scf_ref, o_ref, *, K):
    def kstep(k, acc):
        return jnp.maximum(acc, scf_ref[k])
    o_ref[...] = jax.lax.fori_loop(
        0, K, kstep, jnp.full(o_ref.shape, -jnp.inf, F32))


def _scmax_pallas(scf_g, B=512):
    K, Np, C = scf_g.shape
    return pl.pallas_call(
        functools.partial(_scmax_body, K=K),
        grid=(Np // B,),
        in_specs=[pl.BlockSpec((K, B, C), lambda i: (0, i, 0))],
        out_specs=pl.BlockSpec((B, C), lambda i: (i, 0)),
        out_shape=jax.ShapeDtypeStruct((Np, C), F32),
    )(scf_g)


def _mm_body(x_ref, w_ref, mu_ref, sc_ref, o_ref, *, fuse_act):
    x = x_ref[...]
    if fuse_act:
        x = _lrelu((x - mu_ref[...]) * sc_ref[...])
    o_ref[...] = _dot(x, w_ref[...])


def _mm_pallas(x, w, mu=None, scale=None, B=1024):
    n_in = x.shape[0]
    Np = _ceil_to(n_in, B)
    if Np != n_in:
        x = jnp.pad(x, ((0, Np - n_in), (0, 0)))
    Cin = x.shape[1]
    D = w.shape[1]
    fuse = scale is not None
    if not fuse:
        mu = jnp.zeros((1, Cin), F32)
        scale = jnp.ones((1, Cin), F32)
    else:
        mu = mu.reshape(1, Cin)
        scale = scale.reshape(1, Cin)
    return pl.pallas_call(
        functools.partial(_mm_body, fuse_act=fuse),
        grid=(Np // B,),
        in_specs=[
            pl.BlockSpec((B, Cin), lambda i: (i, 0)),
            pl.BlockSpec((Cin, D), lambda i: (0, 0)),
            pl.BlockSpec((1, Cin), lambda i: (0, 0)),
            pl.BlockSpec((1, Cin), lambda i: (0, 0)),
        ],
        out_specs=pl.BlockSpec((B, D), lambda i: (i, 0)),
        out_shape=jax.ShapeDtypeStruct((Np, D), F32),
    )(x, w, mu, scale)


def _gn_affine(x_valid, gamma, beta, groups=8):
    n, c = x_valid.shape
    gs = c // groups
    xg = x_valid.reshape(n, groups, gs)
    mean = jnp.repeat(xg.mean(axis=(0, 2)), gs)
    var = jnp.repeat(xg.var(axis=(0, 2)), gs)
    scale = jax.lax.rsqrt(var + GN_EPS) * gamma
    safe = jnp.where(scale == 0, 1.0, scale)
    mu_eff = mean - jnp.where(scale == 0, 0.0, beta / safe)
    return mu_eff, scale


def _gather_kmajor(table, neighbors_t, Np):
    K, N = neighbors_t.shape
    idx = neighbors_t if Np == N else jnp.pad(neighbors_t,
                                              ((0, 0), (0, Np - N)))
    Rn = K * Np
    idx2 = idx.reshape(1, Rn).astype(jnp.int32)
    Wd = table.shape[1]
    win = 128
    mesh = plsc.VectorSubcoreMesh(core_axis_name="c", subcore_axis_name="s")
    kw = {}
    if Wd % 128 != 0:
        kw['compiler_params'] = pltpu.CompilerParams(
            use_tc_tiling_on_sc=False)

    @functools.partial(pl.kernel,
                       out_type=jax.ShapeDtypeStruct((Rn, Wd), F32),
                       mesh=mesh, **kw)
    def gk(table_hbm, i_hbm, o_hbm):
        def body(i_vmem, o_vmem):
            pltpu.sync_copy(table_hbm.at[i_vmem.at[0]], o_vmem)

        pltpu.emit_pipeline(
            body,
            grid=(Rn // win,),
            in_specs=[pl.BlockSpec((1, win), index_map=lambda i: (0, i))],
            out_specs=[pl.BlockSpec((win, Wd), index_map=lambda i: (i, 0))],
            core_axis_name=("c", "s"),
            dimension_semantics=(pltpu.PARALLEL,),
        )(i_hbm, o_hbm)

    return gk(table, idx2).reshape(K, Np, Wd)


def _pts_pad(points, Np):
    N = points.shape[0]
    return jnp.pad(points, ((0, Np - N), (0, XP - 3)))


def _m2_const(kpts):
    m2 = jnp.zeros((2 * XP, XP), F32)
    m2 = m2.at[0:3, :KS].set(-2.0 * kpts.T)
    m2 = m2.at[XP:XP + 3, :].set(1.0)
    return m2


def _kn_const(kpts):
    kn = jnp.zeros((1, XP), F32)
    return kn.at[0, :KS].set((kpts * kpts).sum(-1))


def _res_block(p, s_feats, pts_g, q_pts, nb_t, sigma, Np_q, Np_s, strided):
    N_q = nb_t.shape[1]
    N_s = s_feats.shape[0]
    x = _mm_pallas(s_feats, p['u1_W'])[:N_s]
    mu1, sc1 = _gn_affine(x, p['u1_g'], p['u1_b'])
    nf_g = _gather_kmajor(x, nb_t, Np_q)
    kp = _kpconv_pallas(pts_g, q_pts, _m2_const(p['kpts']),
                        _kn_const(p['kpts']), nf_g,
                        mu1.reshape(1, -1), sc1.reshape(1, -1),
                        p['kp_W'], sigma)[:N_q]
    mu2, sc2 = _gn_affine(kp, p['kn_g'], p['kn_b'])
    y = _mm_pallas(kp, p['u2_W'], mu=mu2, scale=sc2)[:N_q]
    mu3, sc3 = _gn_affine(y, p['u2_g'], p['u2_b'])
    y = (y - mu3) * sc3
    if strided:
        scf_g = _gather_kmajor(s_feats, nb_t, Np_q)
        sc = _scmax_pallas(scf_g)[:N_q]
    else:
        sc = s_feats
    if 'sc_W' in p:
        sc = _mm_pallas(sc, p['sc_W'])[:N_q]
        mu4, sc4 = _gn_affine(sc, p['sc_g'], p['sc_b'])
        sc = (sc - mu4) * sc4
    return _lrelu(y + sc)


def kernel(points_0, points_1, points_2, neighbors_0, neighbors_1,
           neighbors_2, subsampling_0, subsampling_1, upsampling_0, params):
    N0 = points_0.shape[0]
    N1 = points_1.shape[0]
    N2 = points_2.shape[0]
    B = 512
    Np0, Np1, Np2 = _ceil_to(N0, B), _ceil_to(N1, B), _ceil_to(N2, B)
    p = params

    pp0 = _pts_pad(points_0, Np0)
    pp1 = _pts_pad(points_1, Np1)
    pp2 = _pts_pad(points_2, Np2)

    nb0_t = neighbors_0.T
    nb1_t = neighbors_1.T
    nb2_t = neighbors_2.T
    ss0_t = subsampling_0.T
    ss1_t = subsampling_1.T

    g_nb0 = _gather_kmajor(pp0[:N0, :], nb0_t, Np0)
    g_ss0 = _gather_kmajor(pp0[:N0, :], ss0_t, Np1)
    g_nb1 = _gather_kmajor(pp1[:N1, :], nb1_t, Np1)
    g_ss1 = _gather_kmajor(pp1[:N1, :], ss1_t, Np2)
    g_nb2 = _gather_kmajor(pp2[:N2, :], nb2_t, Np2)

    w0 = jnp.pad(p['e11']['kp_W'][:, 0, :], ((0, XP - KS), (0, 0)))
    feats = _e11_pallas(g_nb0, pp0, _m2_const(p['e11']['kpts']),
                        _kn_const(p['e11']['kpts']), w0, S0)[:N0]
    mu, sc = _gn_affine(feats, p['e11']['g'], p['e11']['b'])
    feats = _lrelu((feats - mu) * sc)

    feats = _res_block(p['e12'], feats, g_nb0, pp0, nb0_t, S0, Np0, Np0,
                       strided=False)
    feats = _res_block(p['l1_0'], feats, g_ss0, pp1, ss0_t, S0, Np1, Np0,
                       strided=True)
    feats = _res_block(p['l1_1'], feats, g_nb1, pp1, nb1_t, 2 * S0, Np1, Np1,
                       strided=False)
    f1 = _res_block(p['l1_2'], feats, g_nb1, pp1, nb1_t, 2 * S0, Np1, Np1,
                    strided=False)
    feats = _res_block(p['l2_0'], f1, g_ss1, pp2, ss1_t, 2 * S0, Np2, Np1,
                       strided=True)
    feats = _res_block(p['l2_1'], feats, g_nb2, pp2, nb2_t, 4 * S0, Np2, Np2,
                       strided=False)
    f2 = _res_block(p['l2_2'], feats, g_nb2, pp2, nb2_t, 4 * S0, Np2, Np2,
                    strided=False)

    up = jnp.concatenate([f1, f2[upsampling_0[:, 0]]], axis=1)
    f1d = _mm_pallas(up, p['dec0_W'])[:N1]
    mud, scd = _gn_affine(f1d, p['dec0_g'], p['dec0_b'])
    f1d = _lrelu((f1d - mud) * scd)

    d2 = ((points_2[:, None, :] - points_1[None, :, :]) ** 2).sum(-1)
    NCH = _ceil_to(N1, 128) // 128
    d2p = jnp.pad(d2, ((0, 0), (0, NCH * 128 - N1)),
                  constant_values=jnp.inf)
    d2c = d2p.reshape(N2, NCH, 128)
    cmin = d2c.min(axis=2)
    _u1, cidx = jax.lax.top_k(-cmin, 32)
    cand = jnp.take_along_axis(d2c, cidx[:, :, None], axis=1)
    cand = cand.reshape(N2, 32 * 128)
    _u2, j = jax.lax.top_k(-cand, 32)
    idx = cidx[jnp.arange(N2)[:, None], j // 128] * 128 + j % 128
    gx = points_1[idx]
    gf = f1d[idx]
    rel = gx - points_2[:, None, :]
    h = _lrelu(jnp.concatenate([rel, gf], axis=-1) @ p['det_W1']
               + p['det_b1'])
    scores = (h @ p['det_W2'] + p['det_b2'])[..., 0]
    attn = jax.nn.softmax(scores, axis=-1)
    xyz = jnp.einsum('mk,mki->mi', attn, gx)
    dist = jnp.sqrt(((gx - xyz[:, None, :]) ** 2).sum(-1) + 1e-12)
    sigma_out = jnp.einsum('mk,mk->m', attn, dist)[:, None]
    att_feat = jnp.einsum('mk,mkc->mc', attn, gf)
    g = jnp.max(_lrelu(gf @ p['desc_Wg']), axis=1)
    a = att_feat @ p['desc_Wa']
    desc = jnp.concatenate([g, a], axis=-1) @ p['desc_Wo']
    desc = desc / (jnp.linalg.norm(desc, axis=-1, keepdims=True) + 1e-8)
    return (f1d, f2, xyz, sigma_out, desc)
